# baseline (speedup 1.0000x reference)
import functools

import jax
import jax.numpy as jnp
from jax import lax
from jax.experimental import pallas as pl
from jax.experimental.pallas import tpu as pltpu
from jax.experimental.pallas import tpu_sc as plsc

B = 1024
D = 64
LENS = (20, 50, 100, 200)
NC, NS = 2, 16
NW = NC * NS
BPW = B // NW
CHUNK = (16, 8, 4, 2)


def _accumulate_rows(buf, b, n_rows, out_v, out_row, out_off):
    def body(l, accs):
        base = l * D
        return tuple(
            acc + buf[b, pl.ds(base + d * 16, 16)] for d, acc in enumerate(accs)
        )

    zero = jnp.zeros((16,), jnp.float32)
    accs = lax.fori_loop(0, n_rows, body, (zero, zero, zero, zero))
    for d in range(4):
        out_v[out_row, pl.ds(out_off + d * 16, 16)] = accs[d]


def _sc_body(in0, in1, in2, in3, out, b0, b1, b2, b3, out_v):
    wid = lax.axis_index("s") * NC + lax.axis_index("c")
    base = wid * BPW
    bufs = (b0, b1, b2, b3)
    ins = (in0, in1, in2, in3)
    for i, (inp, buf, nb, L) in enumerate(zip(ins, bufs, CHUNK, LENS)):
        for c0 in range(0, BPW, nb):
            pltpu.sync_copy(inp.at[pl.ds(base + c0, nb)], buf)
            for b in range(nb):
                _accumulate_rows(buf, b, L, out_v, c0 + b, i * D)
    pltpu.sync_copy(out_v, out.at[pl.ds(base, BPW)])


def _build_sc_call():
    mesh = plsc.VectorSubcoreMesh(
        core_axis_name="c", subcore_axis_name="s", num_cores=NC, num_subcores=NS
    )
    scratch = [
        pltpu.VMEM((CHUNK[i], LENS[i] * D), jnp.float32) for i in range(4)
    ] + [pltpu.VMEM((BPW, 4 * D), jnp.float32)]
    return pl.kernel(
        _sc_body,
        out_type=jax.ShapeDtypeStruct((B, 4 * D), jnp.float32),
        mesh=mesh,
        scratch_types=scratch,
    )


def kernel(inputs_0, inputs_1, inputs_2, inputs_3, sum_dim, concat_mode,
           keep_dims, cat_axis, is_cat):
    ins = [
        t.reshape(B, L * D)
        for t, L in zip((inputs_0, inputs_1, inputs_2, inputs_3), LENS)
    ]
    out = _build_sc_call()(*ins)
    return out.reshape(B, 4, D)


# speedup vs baseline: 1.2124x; 1.2124x over previous
import jax
import jax.numpy as jnp
from jax import lax
from jax.experimental import pallas as pl
from jax.experimental.pallas import tpu as pltpu
from jax.experimental.pallas import tpu_sc as plsc

B = 1024
D = 64
LENS = (20, 50, 100, 200)
NC, NS = 2, 16
NW = NC * NS
BPW = B // NW
NB = (8, 4, 2, 1)
CHUNKS = [
    (i, c0, (c0 // NB[i]) % 2)
    for i in range(4)
    for c0 in range(0, BPW, NB[i])
]


def _accumulate(buf, b, n_rows, out_v, out_row, out_off):
    zero = jnp.zeros((16,), jnp.float32)

    @plsc.parallel_loop(0, n_rows, step=1, unroll=5, carry=(zero,) * 4)
    def accs(l, accs):
        base = l * D
        return tuple(
            acc + buf[b, pl.ds(base + d * 16, 16)] for d, acc in enumerate(accs)
        )

    for d in range(4):
        out_v[out_row, pl.ds(out_off + d * 16, 16)] = accs[d]


def _sc_body(in0, in1, in2, in3, out, *scratch):
    bufs = [scratch[2 * i:2 * i + 2] for i in range(4)]
    out_v = scratch[8]
    sems = [scratch[9 + 2 * i:11 + 2 * i] for i in range(4)]
    ins = (in0, in1, in2, in3)
    wid = lax.axis_index("s") * NC + lax.axis_index("c")
    base = wid * BPW

    def issue(j):
        i, c0, s = CHUNKS[j]
        return pltpu.async_copy(
            ins[i].at[pl.ds(base + c0, NB[i])], bufs[i][s], sems[i][s]
        )

    n = len(CHUNKS)
    descs = [None] * n
    descs[0] = issue(0)
    descs[1] = issue(1)
    for j, (i, c0, s) in enumerate(CHUNKS):
        descs[j].wait()
        for b in range(NB[i]):
            _accumulate(bufs[i][s], b, LENS[i], out_v, c0 + b, i * D)
        if j + 2 < n:
            descs[j + 2] = issue(j + 2)
    pltpu.sync_copy(out_v, out.at[pl.ds(base, BPW)])


def _build_sc_call():
    mesh = plsc.VectorSubcoreMesh(
        core_axis_name="c", subcore_axis_name="s", num_cores=NC, num_subcores=NS
    )
    scratch = [
        pltpu.VMEM((NB[i], LENS[i] * D), jnp.float32)
        for i in range(4) for _ in range(2)
    ] + [pltpu.VMEM((BPW, 4 * D), jnp.float32)] + [
        pltpu.SemaphoreType.DMA for _ in range(8)
    ]
    return pl.kernel(
        _sc_body,
        out_type=jax.ShapeDtypeStruct((B, 4 * D), jnp.float32),
        mesh=mesh,
        scratch_types=scratch,
    )


def kernel(inputs_0, inputs_1, inputs_2, inputs_3, sum_dim, concat_mode,
           keep_dims, cat_axis, is_cat):
    ins = [
        t.reshape(B, L * D)
        for t, L in zip((inputs_0, inputs_1, inputs_2, inputs_3), LENS)
    ]
    out = _build_sc_call()(*ins)
    return out.reshape(B, 4, D)


# speedup vs baseline: 2.9011x; 2.3928x over previous
import jax
import jax.numpy as jnp
from jax import lax
from jax.experimental import pallas as pl
from jax.experimental.pallas import tpu as pltpu
from jax.experimental.pallas import tpu_sc as plsc

B = 1024
D = 64
LENS = (20, 50, 100, 200)
NC, NS = 2, 16
NW = NC * NS
LC = 16
NBUF = 3

CHUNKS = []
for _i, _L in enumerate(LENS):
    _l0 = 0
    while _l0 < _L:
        _lc = min(LC, _L - _l0)
        CHUNKS.append((_i, _l0, _lc, _l0 == 0, _l0 + _lc == _L))
        _l0 += _lc


def _unroll_for(lc):
    for u in (4, 2):
        if lc % u == 0:
            return u
    return 1


def _sc_body(x0, x1, x2, x3, out, *scratch):
    bufs = scratch[:NBUF]
    acc = scratch[NBUF]
    sems = scratch[NBUF + 1:]
    ins = (x0, x1, x2, x3)
    w = lax.axis_index("s") * NC + lax.axis_index("c")
    dg = w // 4
    bp = w % 4
    d0 = dg * 8
    b0 = bp * 256

    def issue(j):
        i, l0, lc, _, _ = CHUNKS[j]
        return pltpu.async_copy(
            ins[i].at[pl.ds(l0, lc), pl.ds(d0, 8), pl.ds(b0, 256)],
            bufs[j % NBUF].at[pl.ds(0, lc)],
            sems[j % NBUF],
        )

    n = len(CHUNKS)
    descs = [None] * n
    for j in range(min(NBUF, n)):
        descs[j] = issue(j)
    for j, (i, l0, lc, first, last) in enumerate(CHUNKS):
        descs[j].wait()
        buf = bufs[j % NBUF]
        unroll = _unroll_for(lc)

        def col_body(c, carry):
            s = c // 16
            koff = (c % 16) * 16
            zero = jnp.zeros((16,), jnp.float32)

            @plsc.parallel_loop(0, lc, step=1, unroll=unroll, carry=zero)
            def colsum(l, a):
                return a + buf[l, s, pl.ds(koff, 16)]

            if first:
                acc[s, pl.ds(koff, 16)] = colsum
            else:
                acc[s, pl.ds(koff, 16)] = acc[s, pl.ds(koff, 16)] + colsum
            return carry

        lax.fori_loop(0, 128, col_body, jnp.int32(0))
        if last:
            pltpu.sync_copy(acc, out.at[i, dg, bp])
        if j + NBUF < n:
            descs[j + NBUF] = issue(j + NBUF)


def _build_sc_call():
    mesh = plsc.VectorSubcoreMesh(
        core_axis_name="c", subcore_axis_name="s", num_cores=NC, num_subcores=NS
    )
    scratch = [pltpu.VMEM((LC, 8, 256), jnp.float32) for _ in range(NBUF)]
    scratch += [pltpu.VMEM((8, 256), jnp.float32)]
    scratch += [pltpu.SemaphoreType.DMA for _ in range(NBUF)]
    return pl.kernel(
        _sc_body,
        out_type=jax.ShapeDtypeStruct((4, 8, 4, 8, 256), jnp.float32),
        mesh=mesh,
        scratch_types=scratch,
        compiler_params=pltpu.CompilerParams(use_tc_tiling_on_sc=True),
    )


def kernel(inputs_0, inputs_1, inputs_2, inputs_3, sum_dim, concat_mode,
           keep_dims, cat_axis, is_cat):
    xt = [
        jnp.transpose(t, (1, 2, 0))
        for t in (inputs_0, inputs_1, inputs_2, inputs_3)
    ]
    out5 = _build_sc_call()(*xt)
    return out5.transpose(2, 4, 0, 1, 3).reshape(B, 4, D)


# speedup vs baseline: 3.7784x; 1.3024x over previous
import functools

import jax
import jax.numpy as jnp
from jax import lax
from jax.experimental import pallas as pl
from jax.experimental.pallas import tpu as pltpu
from jax.experimental.pallas import tpu_sc as plsc

B = 1024
D = 64
LENS = (20, 50, 100, 200)
NC, NS = 2, 16
NW = NC * NS
LC = 16
NBUF = 3

SC_INPUTS = (0, 1, 2)
TC_INPUTS = (3,)

CHUNKS = []
for _slot, _i in enumerate(SC_INPUTS):
    _L = LENS[_i]
    _l0 = 0
    while _l0 < _L:
        _lc = min(LC, _L - _l0)
        CHUNKS.append((_slot, _l0, _lc, _l0 == 0, _l0 + _lc == _L))
        _l0 += _lc


def _unroll_for(lc):
    for u in (4, 2):
        if lc % u == 0:
            return u
    return 1


def _sc_body(*refs):
    n_in = len(SC_INPUTS)
    ins = refs[:n_in]
    out = refs[n_in]
    scratch = refs[n_in + 1:]
    bufs = scratch[:NBUF]
    acc = scratch[NBUF]
    sems = scratch[NBUF + 1:]
    w = lax.axis_index("s") * NC + lax.axis_index("c")
    dg = w // 4
    bp = w % 4
    d0 = dg * 8
    b0 = bp * 256

    def issue(j):
        slot, l0, lc, _, _ = CHUNKS[j]
        return pltpu.async_copy(
            ins[slot].at[pl.ds(l0, lc), pl.ds(d0, 8), pl.ds(b0, 256)],
            bufs[j % NBUF].at[pl.ds(0, lc)],
            sems[j % NBUF],
        )

    n = len(CHUNKS)
    descs = [None] * n
    for j in range(min(NBUF, n)):
        descs[j] = issue(j)
    for j, (slot, l0, lc, first, last) in enumerate(CHUNKS):
        descs[j].wait()
        buf = bufs[j % NBUF]
        unroll = _unroll_for(lc)

        def col_body(c, carry):
            s = c // 16
            koff = (c % 16) * 16
            zero = jnp.zeros((16,), jnp.float32)

            @plsc.parallel_loop(0, lc, step=1, unroll=unroll, carry=zero)
            def colsum(l, a):
                return a + buf[l, s, pl.ds(koff, 16)]

            if first:
                acc[s, pl.ds(koff, 16)] = colsum
            else:
                acc[s, pl.ds(koff, 16)] = acc[s, pl.ds(koff, 16)] + colsum
            return carry

        lax.fori_loop(0, 128, col_body, jnp.int32(0))
        if last:
            pltpu.sync_copy(acc, out.at[slot, dg, bp])
        if j + NBUF < n:
            descs[j + NBUF] = issue(j + NBUF)


def _build_sc_call():
    mesh = plsc.VectorSubcoreMesh(
        core_axis_name="c", subcore_axis_name="s", num_cores=NC, num_subcores=NS
    )
    scratch = [pltpu.VMEM((LC, 8, 256), jnp.float32) for _ in range(NBUF)]
    scratch += [pltpu.VMEM((8, 256), jnp.float32)]
    scratch += [pltpu.SemaphoreType.DMA for _ in range(NBUF)]
    return pl.kernel(
        _sc_body,
        out_type=jax.ShapeDtypeStruct((len(SC_INPUTS), 8, 4, 8, 256), jnp.float32),
        mesh=mesh,
        scratch_types=scratch,
        compiler_params=pltpu.CompilerParams(use_tc_tiling_on_sc=True),
    )


def _tc_body(x_ref, o_ref):
    l = pl.program_id(0)

    @pl.when(l == 0)
    def _():
        o_ref[...] = jnp.zeros_like(o_ref)

    o_ref[...] += jnp.sum(x_ref[...], axis=0)


def _tc_sum(xt, lc):
    L = xt.shape[0]
    return pl.pallas_call(
        _tc_body,
        grid=(L // lc,),
        in_specs=[pl.BlockSpec((lc, D, B), lambda l: (l, 0, 0))],
        out_specs=pl.BlockSpec((D, B), lambda l: (0, 0)),
        out_shape=jax.ShapeDtypeStruct((D, B), jnp.float32),
        compiler_params=pltpu.CompilerParams(
            dimension_semantics=("arbitrary",),
        ),
    )(xt)


def kernel(inputs_0, inputs_1, inputs_2, inputs_3, sum_dim, concat_mode,
           keep_dims, cat_axis, is_cat):
    xs = (inputs_0, inputs_1, inputs_2, inputs_3)
    xt = [jnp.transpose(t, (1, 2, 0)) for t in xs]

    sc_out = _build_sc_call()(*[xt[i] for i in SC_INPUTS])
    sc_part = sc_out.transpose(2, 4, 0, 1, 3).reshape(B, len(SC_INPUTS), D)

    tc_parts = [
        _tc_sum(xt[i], 25).T.reshape(B, 1, D) for i in TC_INPUTS
    ]
    return jnp.concatenate([sc_part] + tc_parts, axis=1)


# speedup vs baseline: 3.7860x; 1.0020x over previous
import functools

import jax
import jax.numpy as jnp
from jax import lax
from jax.experimental import pallas as pl
from jax.experimental.pallas import tpu as pltpu
from jax.experimental.pallas import tpu_sc as plsc

B = 1024
D = 64
LENS = (20, 50, 100, 200)
NC, NS = 2, 16
NW = NC * NS
LC = 16
NBUF = 3

SC_INPUTS = (0, 1, 2)
TC_INPUTS = (3,)

CHUNKS = []
for _slot, _i in enumerate(SC_INPUTS):
    _L = LENS[_i]
    _l0 = 0
    while _l0 < _L:
        _lc = min(LC, _L - _l0)
        CHUNKS.append((_slot, _l0, _lc, _l0 == 0, _l0 + _lc == _L))
        _l0 += _lc


def _unroll_for(lc):
    for u in (4, 2):
        if lc % u == 0:
            return u
    return 1


def _sc_body(*refs):
    n_in = len(SC_INPUTS)
    ins = refs[:n_in]
    out = refs[n_in]
    scratch = refs[n_in + 1:]
    bufs = scratch[:NBUF]
    acc = scratch[NBUF]
    sems = scratch[NBUF + 1:]
    w = lax.axis_index("s") * NC + lax.axis_index("c")
    dg = w // 4
    bp = w % 4
    d0 = dg * 8
    b0 = bp * 256

    def issue(j):
        slot, l0, lc, _, _ = CHUNKS[j]
        return pltpu.async_copy(
            ins[slot].at[pl.ds(l0, lc), pl.ds(d0, 8), pl.ds(b0, 256)],
            bufs[j % NBUF].at[pl.ds(0, lc)],
            sems[j % NBUF],
        )

    n = len(CHUNKS)
    descs = [None] * n
    for j in range(min(NBUF, n)):
        descs[j] = issue(j)
    for j, (slot, l0, lc, first, last) in enumerate(CHUNKS):
        descs[j].wait()
        buf = bufs[j % NBUF]
        unroll = _unroll_for(lc)

        def col_body(c, carry):
            s = c // 16
            koff = (c % 16) * 16
            zero = jnp.zeros((16,), jnp.float32)

            @plsc.parallel_loop(0, lc, step=1, unroll=unroll, carry=zero)
            def colsum(l, a):
                return a + buf[l, s, pl.ds(koff, 16)]

            if first:
                acc[s, pl.ds(koff, 16)] = colsum
            else:
                acc[s, pl.ds(koff, 16)] = acc[s, pl.ds(koff, 16)] + colsum
            return carry

        lax.fori_loop(0, 128, col_body, jnp.int32(0))
        if last:
            pltpu.sync_copy(acc, out.at[slot, dg, bp])
        if j + NBUF < n:
            descs[j + NBUF] = issue(j + NBUF)


def _build_sc_call():
    mesh = plsc.VectorSubcoreMesh(
        core_axis_name="c", subcore_axis_name="s", num_cores=NC, num_subcores=NS
    )
    scratch = [pltpu.VMEM((LC, 8, 256), jnp.float32) for _ in range(NBUF)]
    scratch += [pltpu.VMEM((8, 256), jnp.float32)]
    scratch += [pltpu.SemaphoreType.DMA for _ in range(NBUF)]
    return pl.kernel(
        _sc_body,
        out_type=jax.ShapeDtypeStruct((len(SC_INPUTS), 8, 4, 8, 256), jnp.float32),
        mesh=mesh,
        scratch_types=scratch,
        compiler_params=pltpu.CompilerParams(
            use_tc_tiling_on_sc=True, skip_device_barrier=True
        ),
    )


def _tc_body(x_ref, o_ref):
    l = pl.program_id(0)

    @pl.when(l == 0)
    def _():
        o_ref[...] = jnp.zeros_like(o_ref)

    o_ref[...] += jnp.sum(x_ref[...], axis=0)


def _tc_sum(xt, lc):
    L = xt.shape[0]
    return pl.pallas_call(
        _tc_body,
        grid=(L // lc,),
        in_specs=[pl.BlockSpec((lc, D, B), lambda l: (l, 0, 0))],
        out_specs=pl.BlockSpec((D, B), lambda l: (0, 0)),
        out_shape=jax.ShapeDtypeStruct((D, B), jnp.float32),
        compiler_params=pltpu.CompilerParams(
            dimension_semantics=("arbitrary",), skip_device_barrier=True,
        ),
    )(xt)


def kernel(inputs_0, inputs_1, inputs_2, inputs_3, sum_dim, concat_mode,
           keep_dims, cat_axis, is_cat):
    xs = (inputs_0, inputs_1, inputs_2, inputs_3)
    xt = [jnp.transpose(t, (1, 2, 0)) for t in xs]

    sc_out = _build_sc_call()(*[xt[i] for i in SC_INPUTS])
    sc_part = sc_out.transpose(2, 4, 0, 1, 3).reshape(B, len(SC_INPUTS), D)

    tc_parts = [
        _tc_sum(xt[i], 25).T.reshape(B, 1, D) for i in TC_INPUTS
    ]
    return jnp.concatenate([sc_part] + tc_parts, axis=1)


# speedup vs baseline: 6.8085x; 1.7983x over previous
import jax
import jax.numpy as jnp
from jax.experimental import pallas as pl
from jax.experimental.pallas import tpu as pltpu

B = 1024
D = 64
LENS = (20, 50, 100, 200)
GRID = 10
LCS = tuple(L // GRID for L in LENS)


def _tc_body(x0, x1, x2, x3, o_ref):
    g = pl.program_id(0)

    @pl.when(g == 0)
    def _():
        o_ref[...] = jnp.zeros_like(o_ref)

    for i, x in enumerate((x0, x1, x2, x3)):
        o_ref[i, :, :] += jnp.sum(x[...], axis=0)


def kernel(inputs_0, inputs_1, inputs_2, inputs_3, sum_dim, concat_mode,
           keep_dims, cat_axis, is_cat):
    xs = (inputs_0, inputs_1, inputs_2, inputs_3)
    xt = [jnp.transpose(t, (1, 2, 0)) for t in xs]
    out = pl.pallas_call(
        _tc_body,
        grid=(GRID,),
        in_specs=[
            pl.BlockSpec((lc, D, B), lambda g, _lc=lc: (g, 0, 0))
            for lc in LCS
        ],
        out_specs=pl.BlockSpec((4, D, B), lambda g: (0, 0, 0)),
        out_shape=jax.ShapeDtypeStruct((4, D, B), jnp.float32),
        compiler_params=pltpu.CompilerParams(
            dimension_semantics=("arbitrary",),
        ),
    )(*xt)
    return out.transpose(2, 0, 1)
